# Initial kernel scaffold; baseline (speedup 1.0000x reference)
import jax
import jax.numpy as jnp
from jax.experimental import pallas as pl


def kernel(words, chars, embed_weight):
    raise NotImplementedError("write your pallas kernel here")


# speedup vs baseline: 15.5154x; 15.5154x over previous
import functools

import jax
import jax.numpy as jnp
from jax import lax
from jax.experimental import pallas as pl
from jax.experimental.pallas import tpu as pltpu
from jax.experimental.pallas import tpu_sc as plsc

_L = 16
_NW = 32


def _pooled_embed(chars_flat, table, BW, C, D, V):
    per_w = BW // _NW
    CH = 200
    n_chunks = per_w // CH
    mesh = plsc.VectorSubcoreMesh(core_axis_name="c", subcore_axis_name="s")

    @functools.partial(
        pl.kernel,
        mesh=mesh,
        out_type=jax.ShapeDtypeStruct((BW, D), jnp.float32),
        scratch_types=[
            pltpu.VMEM((V, D), jnp.float32),
            pltpu.VMEM((CH * C,), jnp.int32),
            pltpu.VMEM((CH, D), jnp.float32),
        ],
    )
    def k(chars_hbm, table_hbm, out_hbm, table_v, idx_v, out_v):
        wid = lax.axis_index("s") * 2 + lax.axis_index("c")
        base = wid * per_w
        pltpu.sync_copy(table_hbm, table_v)

        @pl.loop(0, n_chunks)
        def _(chunk):
            w0 = base + chunk * CH
            pltpu.sync_copy(chars_hbm.at[pl.ds(w0 * C, CH * C)], idx_v)

            @pl.loop(0, CH)
            def _(w):
                idxs = idx_v[pl.ds(w * C, C)]
                idx0 = idxs[0]
                accs = [table_v[idx0, pl.ds(g * _L, _L)] for g in range(D // _L)]
                for c in range(1, C):
                    idx = idxs[c]
                    for g in range(D // _L):
                        row = table_v[idx, pl.ds(g * _L, _L)]
                        accs[g] = jnp.maximum(accs[g], row)
                for g in range(D // _L):
                    out_v[w, pl.ds(g * _L, _L)] = accs[g]

            pltpu.sync_copy(out_v, out_hbm.at[pl.ds(w0, CH)])

    return k(chars_flat, table)


def kernel(words, chars, embed_weight):
    B, W, C = chars.shape
    V, D = embed_weight.shape
    BW = B * W
    chars_flat = chars.reshape(BW * C).astype(jnp.int32)
    pooled = _pooled_embed(chars_flat, embed_weight, BW, C, D, V)
    return pooled.reshape(B, W, D)
